# baseline (speedup 1.0000x reference)
import functools

import jax
import jax.numpy as jnp
from jax import lax
from jax.experimental import pallas as pl
from jax.experimental.pallas import tpu as pltpu
from jax.experimental.pallas import tpu_sc as plsc

N = 10000
NP = 10240
E = 320000
D = 128
NC = 2
NS = 16
NW = NC * NS
E_PER_W = E // NW
B = 64
NCHF = E_PER_W // B
TAIL = E_PER_W - NCHF * B
NSLOT = 5
NTRIP = NCHF // NSLOT
NREM = NCHF - NTRIP * NSLOT
DB = 128
DSLOT = 2
DNCHF = E_PER_W // DB
DTAIL = E_PER_W - DNCHF * DB
DNTRIP = DNCHF // DSLOT
ROWS_PER_S = NP // NS

_MESH = plsc.VectorSubcoreMesh(core_axis_name="c", subcore_axis_name="s")


@functools.partial(
    pl.kernel,
    mesh=_MESH,
    out_type=jax.ShapeDtypeStruct((NC, 1, NP), jnp.float32),
    scratch_types=[
        [pltpu.VMEM((DB,), jnp.int32) for _ in range(DSLOT)],
        pltpu.VMEM((DTAIL,), jnp.int32),
        pltpu.VMEM((DB,), jnp.float32),
        pltpu.MemorySpace.VMEM_SHARED((NP,), jnp.float32),
        [pltpu.SemaphoreType.DMA for _ in range(DSLOT)],
        [pltpu.SemaphoreType.DMA for _ in range(DSLOT)],
    ],
)
def _deg_kernel(dst_hbm, zeros_hbm, ones_hbm, out_hbm, dbufs, dtail,
                ones_buf, acc, isems, ssems):
    c = lax.axis_index("c")
    s = lax.axis_index("s")
    w = c * NS + s
    base = w * E_PER_W

    pltpu.sync_copy(ones_hbm, ones_buf)
    n0 = s * ROWS_PER_S
    pltpu.sync_copy(zeros_hbm.at[0, pl.ds(n0, ROWS_PER_S)],
                    acc.at[pl.ds(n0, ROWS_PER_S)])
    plsc.subcore_barrier()

    def trip(t, _):
        j = t * DSLOT
        ds_ = []
        for bslot in range(DSLOT):
            @pl.when(t > 0)
            def _drain(bslot=bslot):
                pltpu.make_async_copy(ones_buf, acc.at[dbufs[bslot]],
                                      ssems[bslot]).wait()
            ds_.append(pltpu.async_copy(
                dst_hbm.at[pl.ds(base + (j + bslot) * DB, DB)],
                dbufs[bslot], isems[bslot]))
        for bslot in range(DSLOT):
            ds_[bslot].wait()
            pltpu.async_copy(ones_buf, acc.at[dbufs[bslot]], ssems[bslot],
                             add=True)
        return 0

    lax.fori_loop(0, DNTRIP, trip, 0)
    for bslot in range(DSLOT):
        pltpu.make_async_copy(ones_buf, acc.at[dbufs[bslot]],
                              ssems[bslot]).wait()
    if DTAIL:
        pltpu.sync_copy(dst_hbm.at[pl.ds(base + DNCHF * DB, DTAIL)], dtail)
        pltpu.sync_copy(ones_buf.at[pl.ds(0, DTAIL)],
                        acc.at[dtail], add=True)
    plsc.subcore_barrier()
    pltpu.sync_copy(acc.at[pl.ds(n0, ROWS_PER_S)],
                    out_hbm.at[c, 0, pl.ds(n0, ROWS_PER_S)])


@functools.partial(
    pl.kernel,
    mesh=_MESH,
    out_type=jax.ShapeDtypeStruct((NC, NP, D), jnp.float32),
    scratch_types=[
        [pltpu.VMEM((B,), jnp.int32) for _ in range(NSLOT)],
        [pltpu.VMEM((B,), jnp.int32) for _ in range(NSLOT)],
        pltpu.VMEM((TAIL,), jnp.int32),
        pltpu.VMEM((TAIL,), jnp.int32),
        [pltpu.VMEM((B, D), jnp.float32) for _ in range(NSLOT)],
        pltpu.VMEM((TAIL, D), jnp.float32),
        pltpu.MemorySpace.VMEM_SHARED((NP, D), jnp.float32),
        [pltpu.SemaphoreType.DMA for _ in range(NSLOT)],
        [pltpu.SemaphoreType.DMA for _ in range(NSLOT)],
        [pltpu.SemaphoreType.DMA for _ in range(NSLOT)],
        [pltpu.SemaphoreType.DMA for _ in range(NSLOT)],
    ],
)
def _prop_kernel(x_hbm, src_hbm, dst_hbm, zeros_hbm, out_hbm,
                 sbufs, dbufs, stail, dtail, rows, rtail, acc,
                 gsems, ssems, isems, jsems):
    c = lax.axis_index("c")
    s = lax.axis_index("s")
    w = c * NS + s
    r0 = s * ROWS_PER_S
    base = w * E_PER_W
    pltpu.sync_copy(zeros_hbm.at[pl.ds(r0, ROWS_PER_S)],
                    acc.at[pl.ds(r0, ROWS_PER_S)])
    plsc.subcore_barrier()

    def chunk_copies(j, bslot):
        d = pltpu.async_copy(dst_hbm.at[pl.ds(base + j * B, B)],
                             dbufs[bslot], isems[bslot])
        sj = pltpu.async_copy(src_hbm.at[pl.ds(base + j * B, B)],
                              sbufs[bslot], jsems[bslot])
        return d, sj

    def trip(t, _):
        j = t * NSLOT
        ds_ = []
        for bslot in range(NSLOT):
            @pl.when(t > 0)
            def _drain(bslot=bslot):
                pltpu.make_async_copy(rows[bslot], acc.at[dbufs[bslot]],
                                      ssems[bslot]).wait()
            ds_.append(chunk_copies(j + bslot, bslot))
        gs_ = []
        for bslot in range(NSLOT):
            ds_[bslot][1].wait()
            gs_.append(pltpu.async_copy(x_hbm.at[sbufs[bslot]],
                                        rows[bslot], gsems[bslot]))
        for bslot in range(NSLOT):
            ds_[bslot][0].wait()
            gs_[bslot].wait()
            pltpu.async_copy(rows[bslot], acc.at[dbufs[bslot]],
                             ssems[bslot], add=True)
        return 0

    lax.fori_loop(0, NTRIP, trip, 0)
    for bslot in range(NSLOT):
        pltpu.make_async_copy(rows[bslot], acc.at[dbufs[bslot]],
                              ssems[bslot]).wait()
    for r in range(NREM):
        j = NTRIP * NSLOT + r
        d, sj = chunk_copies(j, r)
        sj.wait()
        pltpu.async_copy(x_hbm.at[sbufs[r]], rows[r], gsems[r]).wait()
        d.wait()
        pltpu.sync_copy(rows[r], acc.at[dbufs[r]], add=True)
    if TAIL:
        pltpu.sync_copy(dst_hbm.at[pl.ds(base + NCHF * B, TAIL)], dtail)
        pltpu.sync_copy(src_hbm.at[pl.ds(base + NCHF * B, TAIL)], stail)
        pltpu.async_copy(x_hbm.at[stail], rtail, gsems[0]).wait()
        pltpu.sync_copy(rtail, acc.at[dtail], add=True)
    plsc.subcore_barrier()
    pltpu.sync_copy(acc.at[pl.ds(r0, ROWS_PER_S)],
                    out_hbm.at[c, pl.ds(r0, ROWS_PER_S)])


ROW_BLK = 2000


def _t1_body(d_ref, f_ref, x1_ref):
    nrm = lax.rsqrt(jnp.maximum(d_ref[0] + d_ref[1], 1.0))
    x1_ref[...] = f_ref[...] * nrm


def _t2_body(d_ref, y_ref, f_ref, w0_ref, w1_ref, w2_ref, b_ref,
             x2_ref, r1_ref):
    nrm = lax.rsqrt(jnp.maximum(d_ref[0] + d_ref[1], 1.0))
    ys = y_ref[0] + y_ref[1]
    tx1 = -(ys * nrm)
    x2_ref[...] = tx1 * nrm
    a = w0_ref[...] - w2_ref[...]
    r1_ref[...] = (jnp.dot(f_ref[...], a, preferred_element_type=jnp.float32)
                   + jnp.dot(tx1, w1_ref[...],
                             preferred_element_type=jnp.float32)
                   + b_ref[...])


def _t3_body(d_ref, y_ref, r1_ref, w2_ref, out_ref):
    nrm = lax.rsqrt(jnp.maximum(d_ref[0] + d_ref[1], 1.0))
    h2 = (y_ref[0] + y_ref[1]) * nrm
    out_ref[...] = r1_ref[...] - 2.0 * jnp.dot(
        h2, w2_ref[...], preferred_element_type=jnp.float32)


_GRID = N // ROW_BLK
_deg_spec = pl.BlockSpec((NC, ROW_BLK, 1), lambda i: (0, i, 0))
_rows_spec = pl.BlockSpec((ROW_BLK, D), lambda i: (i, 0))
_y_spec = pl.BlockSpec((NC, ROW_BLK, D), lambda i: (0, i, 0))
_w_spec = pl.BlockSpec((D, D), lambda i: (0, 0))
_b_spec = pl.BlockSpec((1, D), lambda i: (0, 0))

_t1_call = pl.pallas_call(
    _t1_body,
    grid=(_GRID,),
    in_specs=[_deg_spec, _rows_spec],
    out_specs=_rows_spec,
    out_shape=jax.ShapeDtypeStruct((N, D), jnp.float32),
)

_t2_call = pl.pallas_call(
    _t2_body,
    grid=(_GRID,),
    in_specs=[_deg_spec, _y_spec, _rows_spec, _w_spec, _w_spec, _w_spec,
              _b_spec],
    out_specs=[_rows_spec, _rows_spec],
    out_shape=[jax.ShapeDtypeStruct((N, D), jnp.float32),
               jax.ShapeDtypeStruct((N, D), jnp.float32)],
)

_t3_call = pl.pallas_call(
    _t3_body,
    grid=(_GRID,),
    in_specs=[_deg_spec, _y_spec, _rows_spec, _w_spec],
    out_specs=_rows_spec,
    out_shape=jax.ShapeDtypeStruct((N, D), jnp.float32),
)


def kernel(feat, edge_index, W0, W1, W2, b):
    src = edge_index[0].astype(jnp.int32)
    dst = edge_index[1].astype(jnp.int32)
    zerosd = jnp.zeros((NP, D), jnp.float32)
    zeros1 = jnp.zeros((1, NP), jnp.float32)
    ones1 = jnp.ones((DB,), jnp.float32)
    b2 = b.reshape(1, D)

    degw = _deg_kernel(dst, zeros1, ones1).reshape(NC, NP, 1)
    x1 = _t1_call(degw, feat)
    y1 = _prop_kernel(x1, src, dst, zerosd)
    x2, r1 = _t2_call(degw, y1, feat, W0, W1, W2, b2)
    y2 = _prop_kernel(x2, src, dst, zerosd)
    return _t3_call(degw, y2, r1, W2)


# speedup vs baseline: 1.1573x; 1.1573x over previous
import functools

import jax
import jax.numpy as jnp
from jax import lax
from jax.experimental import pallas as pl
from jax.experimental.pallas import tpu as pltpu
from jax.experimental.pallas import tpu_sc as plsc

N = 10000
NP = 10240
E = 320000
D = 128
NC = 2
NS = 16
NW = NC * NS
E_PER_W = E // NW
B = 64
NCHF = E_PER_W // B
TAIL = E_PER_W - NCHF * B
NSLOT = 4
NTRIP = NCHF // NSLOT
DB = 128
DSLOT = 2
DNCHF = E_PER_W // DB
DTAIL = E_PER_W - DNCHF * DB
DNTRIP = DNCHF // DSLOT
ROWS_PER_S = NP // NS

_MESH = plsc.VectorSubcoreMesh(core_axis_name="c", subcore_axis_name="s")


@functools.partial(
    pl.kernel,
    mesh=_MESH,
    out_type=jax.ShapeDtypeStruct((NC, 1, NP), jnp.float32),
    scratch_types=[
        [pltpu.VMEM((DB,), jnp.int32) for _ in range(DSLOT)],
        pltpu.VMEM((DTAIL,), jnp.int32),
        pltpu.VMEM((DB,), jnp.float32),
        pltpu.MemorySpace.VMEM_SHARED((NP,), jnp.float32),
        [pltpu.SemaphoreType.DMA for _ in range(DSLOT)],
        [pltpu.SemaphoreType.DMA for _ in range(DSLOT)],
    ],
)
def _deg_kernel(dst_hbm, zeros_hbm, ones_hbm, out_hbm, dbufs, dtail,
                ones_buf, acc, isems, ssems):
    c = lax.axis_index("c")
    s = lax.axis_index("s")
    w = c * NS + s
    base = w * E_PER_W

    pltpu.sync_copy(ones_hbm, ones_buf)
    n0 = s * ROWS_PER_S
    pltpu.sync_copy(zeros_hbm.at[0, pl.ds(n0, ROWS_PER_S)],
                    acc.at[pl.ds(n0, ROWS_PER_S)])
    plsc.subcore_barrier()

    def trip(t, _):
        j = t * DSLOT
        ds_ = []
        for bslot in range(DSLOT):
            @pl.when(t > 0)
            def _drain(bslot=bslot):
                pltpu.make_async_copy(ones_buf, acc.at[dbufs[bslot]],
                                      ssems[bslot]).wait()
            ds_.append(pltpu.async_copy(
                dst_hbm.at[pl.ds(base + (j + bslot) * DB, DB)],
                dbufs[bslot], isems[bslot]))
        for bslot in range(DSLOT):
            ds_[bslot].wait()
            pltpu.async_copy(ones_buf, acc.at[dbufs[bslot]], ssems[bslot],
                             add=True)
        return 0

    lax.fori_loop(0, DNTRIP, trip, 0)
    for bslot in range(DSLOT):
        pltpu.make_async_copy(ones_buf, acc.at[dbufs[bslot]],
                              ssems[bslot]).wait()
    if DTAIL:
        pltpu.sync_copy(dst_hbm.at[pl.ds(base + DNCHF * DB, DTAIL)], dtail)
        pltpu.sync_copy(ones_buf.at[pl.ds(0, DTAIL)],
                        acc.at[dtail], add=True)
    plsc.subcore_barrier()
    pltpu.sync_copy(acc.at[pl.ds(n0, ROWS_PER_S)],
                    out_hbm.at[c, 0, pl.ds(n0, ROWS_PER_S)])


@functools.partial(
    pl.kernel,
    mesh=_MESH,
    out_type=jax.ShapeDtypeStruct((NC, NP, D), jnp.float32),
    scratch_types=[
        pltpu.VMEM((E_PER_W,), jnp.int32),
        [pltpu.VMEM((B,), jnp.int32) for _ in range(NSLOT)],
        pltpu.VMEM((TAIL,), jnp.int32),
        [pltpu.VMEM((B, D), jnp.float32) for _ in range(NSLOT)],
        pltpu.VMEM((TAIL, D), jnp.float32),
        pltpu.MemorySpace.VMEM_SHARED((NP, D), jnp.float32),
        [pltpu.SemaphoreType.DMA for _ in range(NSLOT)],
        [pltpu.SemaphoreType.DMA for _ in range(NSLOT)],
        [pltpu.SemaphoreType.DMA for _ in range(NSLOT)],
    ],
)
def _prop_kernel(x_hbm, src_hbm, dst_hbm, zeros_hbm, out_hbm,
                 sall, dbufs, dtail, rows, rtail, acc, gsems, ssems, isems):
    c = lax.axis_index("c")
    s = lax.axis_index("s")
    w = c * NS + s
    r0 = s * ROWS_PER_S
    base = w * E_PER_W
    sidx_cp = pltpu.async_copy(src_hbm.at[pl.ds(base, E_PER_W)], sall,
                               gsems[0])
    pltpu.sync_copy(zeros_hbm.at[pl.ds(r0, ROWS_PER_S)],
                    acc.at[pl.ds(r0, ROWS_PER_S)])
    sidx_cp.wait()
    plsc.subcore_barrier()

    def trip(t, _):
        j = t * NSLOT
        ds_ = []
        gs_ = []
        for bslot in range(NSLOT):
            @pl.when(t > 0)
            def _drain(bslot=bslot):
                pltpu.make_async_copy(rows[bslot], acc.at[dbufs[bslot]],
                                      ssems[bslot]).wait()
            ds_.append(pltpu.async_copy(
                dst_hbm.at[pl.ds(base + (j + bslot) * B, B)],
                dbufs[bslot], isems[bslot]))
            gs_.append(pltpu.async_copy(
                x_hbm.at[sall.at[pl.ds((j + bslot) * B, B)]],
                rows[bslot], gsems[bslot]))
        for bslot in range(NSLOT):
            ds_[bslot].wait()
            gs_[bslot].wait()
            pltpu.async_copy(rows[bslot], acc.at[dbufs[bslot]],
                             ssems[bslot], add=True)
        return 0

    lax.fori_loop(0, NTRIP, trip, 0)
    for bslot in range(NSLOT):
        pltpu.make_async_copy(rows[bslot], acc.at[dbufs[bslot]],
                              ssems[bslot]).wait()
    if TAIL:
        pltpu.sync_copy(dst_hbm.at[pl.ds(base + NCHF * B, TAIL)], dtail)
        pltpu.async_copy(
            x_hbm.at[sall.at[pl.ds(NCHF * B, TAIL)]],
            rtail, gsems[0]).wait()
        pltpu.sync_copy(rtail, acc.at[dtail], add=True)
    plsc.subcore_barrier()
    pltpu.sync_copy(acc.at[pl.ds(r0, ROWS_PER_S)],
                    out_hbm.at[c, pl.ds(r0, ROWS_PER_S)])


ROW_BLK = 2000


def _t1_body(d_ref, f_ref, x1_ref):
    nrm = lax.rsqrt(jnp.maximum(d_ref[0] + d_ref[1], 1.0))
    x1_ref[...] = f_ref[...] * nrm


def _t2_body(d_ref, y_ref, f_ref, w0_ref, w1_ref, w2_ref, b_ref,
             x2_ref, r1_ref):
    nrm = lax.rsqrt(jnp.maximum(d_ref[0] + d_ref[1], 1.0))
    ys = y_ref[0] + y_ref[1]
    tx1 = -(ys * nrm)
    x2_ref[...] = tx1 * nrm
    a = w0_ref[...] - w2_ref[...]
    r1_ref[...] = (jnp.dot(f_ref[...], a, preferred_element_type=jnp.float32)
                   + jnp.dot(tx1, w1_ref[...],
                             preferred_element_type=jnp.float32)
                   + b_ref[...])


def _t3_body(d_ref, y_ref, r1_ref, w2_ref, out_ref):
    nrm = lax.rsqrt(jnp.maximum(d_ref[0] + d_ref[1], 1.0))
    h2 = (y_ref[0] + y_ref[1]) * nrm
    out_ref[...] = r1_ref[...] - 2.0 * jnp.dot(
        h2, w2_ref[...], preferred_element_type=jnp.float32)


_GRID = N // ROW_BLK
_deg_spec = pl.BlockSpec((NC, ROW_BLK, 1), lambda i: (0, i, 0))
_rows_spec = pl.BlockSpec((ROW_BLK, D), lambda i: (i, 0))
_y_spec = pl.BlockSpec((NC, ROW_BLK, D), lambda i: (0, i, 0))
_w_spec = pl.BlockSpec((D, D), lambda i: (0, 0))
_b_spec = pl.BlockSpec((1, D), lambda i: (0, 0))

_t1_call = pl.pallas_call(
    _t1_body,
    grid=(_GRID,),
    in_specs=[_deg_spec, _rows_spec],
    out_specs=_rows_spec,
    out_shape=jax.ShapeDtypeStruct((N, D), jnp.float32),
)

_t2_call = pl.pallas_call(
    _t2_body,
    grid=(_GRID,),
    in_specs=[_deg_spec, _y_spec, _rows_spec, _w_spec, _w_spec, _w_spec,
              _b_spec],
    out_specs=[_rows_spec, _rows_spec],
    out_shape=[jax.ShapeDtypeStruct((N, D), jnp.float32),
               jax.ShapeDtypeStruct((N, D), jnp.float32)],
)

_t3_call = pl.pallas_call(
    _t3_body,
    grid=(_GRID,),
    in_specs=[_deg_spec, _y_spec, _rows_spec, _w_spec],
    out_specs=_rows_spec,
    out_shape=jax.ShapeDtypeStruct((N, D), jnp.float32),
)


def kernel(feat, edge_index, W0, W1, W2, b):
    src = edge_index[0].astype(jnp.int32)
    dst = edge_index[1].astype(jnp.int32)
    zerosd = jnp.zeros((NP, D), jnp.float32)
    zeros1 = jnp.zeros((1, NP), jnp.float32)
    ones1 = jnp.ones((DB,), jnp.float32)
    b2 = b.reshape(1, D)

    degw = _deg_kernel(dst, zeros1, ones1).reshape(NC, NP, 1)
    x1 = _t1_call(degw, feat)
    y1 = _prop_kernel(x1, src, dst, zerosd)
    x2, r1 = _t2_call(degw, y1, feat, W0, W1, W2, b2)
    y2 = _prop_kernel(x2, src, dst, zerosd)
    return _t3_call(degw, y2, r1, W2)


# speedup vs baseline: 1.1624x; 1.0044x over previous
import functools

import jax
import jax.numpy as jnp
from jax import lax
from jax.experimental import pallas as pl
from jax.experimental.pallas import tpu as pltpu
from jax.experimental.pallas import tpu_sc as plsc

N = 10000
NP = 10240
E = 320000
D = 128
NC = 2
NS = 16
NW = NC * NS
E_PER_W = E // NW
B = 64
NCHF = E_PER_W // B
TAIL = E_PER_W - NCHF * B
NSLOT = 4
NTRIP = NCHF // NSLOT
DB = 128
DSLOT = 2
DNCHF = E_PER_W // DB
DTAIL = E_PER_W - DNCHF * DB
DNTRIP = DNCHF // DSLOT
ROWS_PER_S = NP // NS

_MESH = plsc.VectorSubcoreMesh(core_axis_name="c", subcore_axis_name="s")


@functools.partial(
    pl.kernel,
    mesh=_MESH,
    out_type=jax.ShapeDtypeStruct((NC, 1, NP), jnp.float32),
    scratch_types=[
        [pltpu.VMEM((DB,), jnp.int32) for _ in range(DSLOT)],
        pltpu.VMEM((DTAIL,), jnp.int32),
        pltpu.VMEM((DB,), jnp.float32),
        pltpu.MemorySpace.VMEM_SHARED((NP,), jnp.float32),
        [pltpu.SemaphoreType.DMA for _ in range(DSLOT)],
        [pltpu.SemaphoreType.DMA for _ in range(DSLOT)],
    ],
)
def _deg_kernel(dst_hbm, zeros_hbm, ones_hbm, out_hbm, dbufs, dtail,
                ones_buf, acc, isems, ssems):
    c = lax.axis_index("c")
    s = lax.axis_index("s")
    w = c * NS + s
    base = w * E_PER_W

    pltpu.sync_copy(ones_hbm, ones_buf)
    n0 = s * ROWS_PER_S
    pltpu.sync_copy(zeros_hbm.at[0, pl.ds(n0, ROWS_PER_S)],
                    acc.at[pl.ds(n0, ROWS_PER_S)])
    plsc.subcore_barrier()

    def trip(t, _):
        j = t * DSLOT
        ds_ = []
        for bslot in range(DSLOT):
            @pl.when(t > 0)
            def _drain(bslot=bslot):
                pltpu.make_async_copy(ones_buf, acc.at[dbufs[bslot]],
                                      ssems[bslot]).wait()
            ds_.append(pltpu.async_copy(
                dst_hbm.at[pl.ds(base + (j + bslot) * DB, DB)],
                dbufs[bslot], isems[bslot]))
        for bslot in range(DSLOT):
            ds_[bslot].wait()
            pltpu.async_copy(ones_buf, acc.at[dbufs[bslot]], ssems[bslot],
                             add=True)
        return 0

    lax.fori_loop(0, DNTRIP, trip, 0)
    for bslot in range(DSLOT):
        pltpu.make_async_copy(ones_buf, acc.at[dbufs[bslot]],
                              ssems[bslot]).wait()
    if DTAIL:
        pltpu.sync_copy(dst_hbm.at[pl.ds(base + DNCHF * DB, DTAIL)], dtail)
        pltpu.sync_copy(ones_buf.at[pl.ds(0, DTAIL)],
                        acc.at[dtail], add=True)
    plsc.subcore_barrier()
    pltpu.sync_copy(acc.at[pl.ds(n0, ROWS_PER_S)],
                    out_hbm.at[c, 0, pl.ds(n0, ROWS_PER_S)])


@functools.partial(
    pl.kernel,
    mesh=_MESH,
    out_type=jax.ShapeDtypeStruct((NC, NP, D), jnp.float32),
    scratch_types=[
        pltpu.VMEM((E_PER_W,), jnp.int32),
        [pltpu.VMEM((B,), jnp.int32) for _ in range(NSLOT)],
        pltpu.VMEM((TAIL,), jnp.int32),
        [pltpu.VMEM((B, D), jnp.float32) for _ in range(NSLOT)],
        pltpu.VMEM((TAIL, D), jnp.float32),
        pltpu.MemorySpace.VMEM_SHARED((NP, D), jnp.float32),
        [pltpu.SemaphoreType.DMA for _ in range(NSLOT)],
        [pltpu.SemaphoreType.DMA for _ in range(NSLOT)],
        [pltpu.SemaphoreType.DMA for _ in range(NSLOT)],
    ],
)
def _prop_kernel(x_hbm, src_hbm, dst_hbm, zeros_hbm, out_hbm,
                 sall, dbufs, dtail, rows, rtail, acc, gsems, ssems, isems):
    c = lax.axis_index("c")
    s = lax.axis_index("s")
    w = c * NS + s
    r0 = s * ROWS_PER_S
    base = w * E_PER_W
    sidx_cp = pltpu.async_copy(src_hbm.at[pl.ds(base, E_PER_W)], sall,
                               gsems[0])
    pltpu.sync_copy(zeros_hbm.at[pl.ds(r0, ROWS_PER_S)],
                    acc.at[pl.ds(r0, ROWS_PER_S)])
    sidx_cp.wait()
    plsc.subcore_barrier()

    def trip(t, _):
        j = t * NSLOT
        ds_ = []
        gs_ = []
        for bslot in range(NSLOT):
            @pl.when(t > 0)
            def _drain(bslot=bslot):
                pltpu.make_async_copy(rows[bslot], acc.at[dbufs[bslot]],
                                      ssems[bslot]).wait()
            ds_.append(pltpu.async_copy(
                dst_hbm.at[pl.ds(base + (j + bslot) * B, B)],
                dbufs[bslot], isems[bslot]))
            gs_.append(pltpu.async_copy(
                x_hbm.at[sall.at[pl.ds((j + bslot) * B, B)]],
                rows[bslot], gsems[bslot]))
        for bslot in range(NSLOT):
            ds_[bslot].wait()
            gs_[bslot].wait()
            pltpu.async_copy(rows[bslot], acc.at[dbufs[bslot]],
                             ssems[bslot], add=True)
        return 0

    lax.fori_loop(0, NTRIP, trip, 0)
    for bslot in range(NSLOT):
        pltpu.make_async_copy(rows[bslot], acc.at[dbufs[bslot]],
                              ssems[bslot]).wait()
    if TAIL:
        pltpu.sync_copy(dst_hbm.at[pl.ds(base + NCHF * B, TAIL)], dtail)
        pltpu.async_copy(
            x_hbm.at[sall.at[pl.ds(NCHF * B, TAIL)]],
            rtail, gsems[0]).wait()
        pltpu.sync_copy(rtail, acc.at[dtail], add=True)
    plsc.subcore_barrier()
    pltpu.sync_copy(acc.at[pl.ds(r0, ROWS_PER_S)],
                    out_hbm.at[c, pl.ds(r0, ROWS_PER_S)])


ROW_BLK = 5000


def _t1_body(d_ref, f_ref, x1_ref):
    nrm = lax.rsqrt(jnp.maximum(d_ref[0] + d_ref[1], 1.0))
    x1_ref[...] = f_ref[...] * nrm


def _t2_body(d_ref, y_ref, f_ref, w0_ref, w1_ref, w2_ref, b_ref,
             x2_ref, r1_ref):
    nrm = lax.rsqrt(jnp.maximum(d_ref[0] + d_ref[1], 1.0))
    ys = y_ref[0] + y_ref[1]
    tx1 = -(ys * nrm)
    x2_ref[...] = tx1 * nrm
    a = w0_ref[...] - w2_ref[...]
    r1_ref[...] = (jnp.dot(f_ref[...], a, preferred_element_type=jnp.float32)
                   + jnp.dot(tx1, w1_ref[...],
                             preferred_element_type=jnp.float32)
                   + b_ref[...])


def _t3_body(d_ref, y_ref, r1_ref, w2_ref, out_ref):
    nrm = lax.rsqrt(jnp.maximum(d_ref[0] + d_ref[1], 1.0))
    h2 = (y_ref[0] + y_ref[1]) * nrm
    out_ref[...] = r1_ref[...] - 2.0 * jnp.dot(
        h2, w2_ref[...], preferred_element_type=jnp.float32)


_GRID = N // ROW_BLK
_deg_spec = pl.BlockSpec((NC, ROW_BLK, 1), lambda i: (0, i, 0))
_rows_spec = pl.BlockSpec((ROW_BLK, D), lambda i: (i, 0))
_y_spec = pl.BlockSpec((NC, ROW_BLK, D), lambda i: (0, i, 0))
_w_spec = pl.BlockSpec((D, D), lambda i: (0, 0))
_b_spec = pl.BlockSpec((1, D), lambda i: (0, 0))

_t1_call = pl.pallas_call(
    _t1_body,
    grid=(_GRID,),
    in_specs=[_deg_spec, _rows_spec],
    out_specs=_rows_spec,
    out_shape=jax.ShapeDtypeStruct((N, D), jnp.float32),
)

_t2_call = pl.pallas_call(
    _t2_body,
    grid=(_GRID,),
    in_specs=[_deg_spec, _y_spec, _rows_spec, _w_spec, _w_spec, _w_spec,
              _b_spec],
    out_specs=[_rows_spec, _rows_spec],
    out_shape=[jax.ShapeDtypeStruct((N, D), jnp.float32),
               jax.ShapeDtypeStruct((N, D), jnp.float32)],
)

_t3_call = pl.pallas_call(
    _t3_body,
    grid=(_GRID,),
    in_specs=[_deg_spec, _y_spec, _rows_spec, _w_spec],
    out_specs=_rows_spec,
    out_shape=jax.ShapeDtypeStruct((N, D), jnp.float32),
)


def kernel(feat, edge_index, W0, W1, W2, b):
    src = edge_index[0].astype(jnp.int32)
    dst = edge_index[1].astype(jnp.int32)
    zerosd = jnp.zeros((NP, D), jnp.float32)
    zeros1 = jnp.zeros((1, NP), jnp.float32)
    ones1 = jnp.ones((DB,), jnp.float32)
    b2 = b.reshape(1, D)

    degw = _deg_kernel(dst, zeros1, ones1).reshape(NC, NP, 1)
    x1 = _t1_call(degw, feat)
    y1 = _prop_kernel(x1, src, dst, zerosd)
    x2, r1 = _t2_call(degw, y1, feat, W0, W1, W2, b2)
    y2 = _prop_kernel(x2, src, dst, zerosd)
    return _t3_call(degw, y2, r1, W2)
